# Initial kernel scaffold; baseline (speedup 1.0000x reference)
import jax
import jax.numpy as jnp
from jax.experimental import pallas as pl


def kernel(inputs_0, inputs_1, inputs_2, inputs_3):
    raise NotImplementedError("write your pallas kernel here")


# speedup vs baseline: 1.1368x; 1.1368x over previous
import jax
import jax.numpy as jnp
from jax.experimental import pallas as pl


_G = 8


def _body(x0, x1, x2, x3, o):
    g = pl.program_id(0)
    s0 = jnp.sum(x0[...], axis=1)
    s1 = jnp.sum(x1[...], axis=1)
    s2 = jnp.sum(x2[...], axis=1)
    s3 = jnp.sum(x3[...], axis=1)
    acc = jnp.concatenate([s0, s1, s2, s3], axis=-1)

    @pl.when(g == 0)
    def _():
        o[...] = acc

    @pl.when(g > 0)
    def _():
        o[...] += acc


def kernel(inputs_0, inputs_1, inputs_2, inputs_3):
    B, D = inputs_0.shape[0], inputs_0.shape[2]
    n = [t.shape[1] for t in (inputs_0, inputs_1, inputs_2, inputs_3)]
    in_specs = [
        pl.BlockSpec((B, ni // _G, D), lambda g: (0, g, 0)) for ni in n
    ]
    return pl.pallas_call(
        _body,
        grid=(_G,),
        in_specs=in_specs,
        out_specs=pl.BlockSpec((B, 4 * D), lambda g: (0, 0)),
        out_shape=jax.ShapeDtypeStruct((B, 4 * D), jnp.float32),
    )(inputs_0, inputs_1, inputs_2, inputs_3)
